# baseline (device time: 48070 ns/iter reference)
import jax
import jax.numpy as jnp
from jax import lax
from jax.experimental import pallas as pl
from jax.experimental.pallas import tpu as pltpu

N_LAYERS = 3


def kernel(x, Win0, Wout0, Win1, Wout1, Win2, Wout2):
    b, d_loc = x.shape
    _, h_loc = Win0.shape

    def body(x_ref, win0_ref, wout0_ref, win1_ref, wout1_ref, win2_ref,
             wout2_ref, out_ref, send_h, recv_h, send_x, recv_x,
             send_sems, recv_sems):
        my_x = lax.axis_index("x")
        my_y = lax.axis_index("y")
        y_peer = (my_x, 1 - my_y)
        x_peer = (1 - my_x, my_y)

        barrier = pltpu.get_barrier_semaphore()
        for peer in (y_peer, x_peer):
            pl.semaphore_signal(
                barrier, inc=1,
                device_id=peer, device_id_type=pl.DeviceIdType.MESH,
            )
        pl.semaphore_wait(barrier, 2)

        win_refs = [win0_ref, win1_ref, win2_ref]
        wout_refs = [wout0_ref, wout1_ref, wout2_ref]

        x_val = x_ref[:, :]
        for layer in range(N_LAYERS):
            e_h = 2 * layer
            e_x = 2 * layer + 1

            send_h[:, :] = jnp.dot(
                x_val, win_refs[layer][:, :],
                preferred_element_type=jnp.float32,
            )
            rdma_h = pltpu.make_async_remote_copy(
                src_ref=send_h,
                dst_ref=recv_h.at[layer],
                send_sem=send_sems.at[e_h],
                recv_sem=recv_sems.at[e_h],
                device_id=y_peer,
                device_id_type=pl.DeviceIdType.MESH,
            )
            rdma_h.start()
            rdma_h.wait()
            h = jnp.maximum(send_h[:, :] + recv_h[layer, :, :], 0.0)

            send_x[:, :] = jnp.dot(
                h, wout_refs[layer][:, :],
                preferred_element_type=jnp.float32,
            )
            rdma_x = pltpu.make_async_remote_copy(
                src_ref=send_x,
                dst_ref=recv_x.at[layer],
                send_sem=send_sems.at[e_x],
                recv_sem=recv_sems.at[e_x],
                device_id=x_peer,
                device_id_type=pl.DeviceIdType.MESH,
            )
            rdma_x.start()
            rdma_x.wait()
            x_val = send_x[:, :] + recv_x[layer, :, :]

        out_ref[:, :] = x_val

    return pl.pallas_call(
        body,
        out_shape=jax.ShapeDtypeStruct((b, d_loc), jnp.float32),
        in_specs=[pl.BlockSpec(memory_space=pltpu.VMEM)] * 7,
        out_specs=pl.BlockSpec(memory_space=pltpu.VMEM),
        scratch_shapes=[
            pltpu.VMEM((b, h_loc), jnp.float32),
            pltpu.VMEM((N_LAYERS, b, h_loc), jnp.float32),
            pltpu.VMEM((b, d_loc), jnp.float32),
            pltpu.VMEM((N_LAYERS, b, d_loc), jnp.float32),
            pltpu.SemaphoreType.DMA((2 * N_LAYERS,)),
            pltpu.SemaphoreType.DMA((2 * N_LAYERS,)),
        ],
        compiler_params=pltpu.CompilerParams(collective_id=0),
    )(x, Win0, Wout0, Win1, Wout1, Win2, Wout2)


# device time: 35436 ns/iter; 1.3565x vs baseline; 1.3565x over previous
import jax
import jax.numpy as jnp
from jax import lax
from jax.experimental import pallas as pl
from jax.experimental.pallas import tpu as pltpu

N_LAYERS = 3


def kernel(x, Win0, Wout0, Win1, Wout1, Win2, Wout2):
    b, d_loc = x.shape
    _, h_loc = Win0.shape

    def body(x_ref, win0_ref, wout0_ref, win1_ref, wout1_ref, win2_ref,
             wout2_ref, out_ref, send_h, recv_h, send_x, recv_x,
             send_sems, recv_sems):
        my_x = lax.axis_index("x")
        my_y = lax.axis_index("y")
        y_peer = (my_x, 1 - my_y)
        x_peer = (1 - my_x, my_y)

        barrier = pltpu.get_barrier_semaphore()
        for peer in (y_peer, x_peer):
            pl.semaphore_signal(
                barrier, inc=1,
                device_id=peer, device_id_type=pl.DeviceIdType.MESH,
            )
        pl.semaphore_wait(barrier, 2)

        win_refs = [win0_ref, win1_ref, win2_ref]
        wout_refs = [wout0_ref, wout1_ref, wout2_ref]

        x_val = x_ref[:, :]
        for layer in range(N_LAYERS):
            e_h = 2 * layer
            e_x = 2 * layer + 1

            ph = jnp.dot(
                x_val, win_refs[layer][:, :],
                preferred_element_type=jnp.float32,
            )
            send_h[:, :] = ph.astype(jnp.bfloat16)
            rdma_h = pltpu.make_async_remote_copy(
                src_ref=send_h,
                dst_ref=recv_h.at[layer],
                send_sem=send_sems.at[e_h],
                recv_sem=recv_sems.at[e_h],
                device_id=y_peer,
                device_id_type=pl.DeviceIdType.MESH,
            )
            rdma_h.start()
            rdma_h.wait()
            h = jnp.maximum(ph + recv_h[layer, :, :].astype(jnp.float32), 0.0)

            px = jnp.dot(
                h, wout_refs[layer][:, :],
                preferred_element_type=jnp.float32,
            )
            send_x[:, :] = px.astype(jnp.bfloat16)
            rdma_x = pltpu.make_async_remote_copy(
                src_ref=send_x,
                dst_ref=recv_x.at[layer],
                send_sem=send_sems.at[e_x],
                recv_sem=recv_sems.at[e_x],
                device_id=x_peer,
                device_id_type=pl.DeviceIdType.MESH,
            )
            rdma_x.start()
            rdma_x.wait()
            x_val = px + recv_x[layer, :, :].astype(jnp.float32)

        out_ref[:, :] = x_val

    return pl.pallas_call(
        body,
        out_shape=jax.ShapeDtypeStruct((b, d_loc), jnp.float32),
        in_specs=[pl.BlockSpec(memory_space=pltpu.VMEM)] * 7,
        out_specs=pl.BlockSpec(memory_space=pltpu.VMEM),
        scratch_shapes=[
            pltpu.VMEM((b, h_loc), jnp.bfloat16),
            pltpu.VMEM((N_LAYERS, b, h_loc), jnp.bfloat16),
            pltpu.VMEM((b, d_loc), jnp.bfloat16),
            pltpu.VMEM((N_LAYERS, b, d_loc), jnp.bfloat16),
            pltpu.SemaphoreType.DMA((2 * N_LAYERS,)),
            pltpu.SemaphoreType.DMA((2 * N_LAYERS,)),
        ],
        compiler_params=pltpu.CompilerParams(collective_id=0),
    )(x, Win0, Wout0, Win1, Wout1, Win2, Wout2)
